# Initial kernel scaffold; baseline (speedup 1.0000x reference)
import jax
import jax.numpy as jnp
from jax.experimental import pallas as pl


def kernel(x, table, W, b):
    raise NotImplementedError("write your pallas kernel here")


# speedup vs baseline: 9.4296x; 9.4296x over previous
import functools

import jax
import jax.numpy as jnp
from jax import lax
from jax.experimental import pallas as pl
from jax.experimental.pallas import tpu as pltpu
from jax.experimental.pallas import tpu_sc as plsc

BATCH = 16384
FIELDS = 26
EMBED_DIM = 32
OUTPUT_DIM = 64
N = BATCH * FIELDS

NUM_CORES = 2
NUM_SUBCORES = 16
NUM_WORKERS = NUM_CORES * NUM_SUBCORES
N_PER_W = N // NUM_WORKERS

IDX_LANE = 128
J_TOTAL = N_PER_W // IDX_LANE
J_PER_CHUNK = 8
CHUNK_ROWS = J_PER_CHUNK * IDX_LANE
N_CHUNKS = J_TOTAL // J_PER_CHUNK


def _make_gather():
    mesh = plsc.VectorSubcoreMesh(core_axis_name="c", subcore_axis_name="s")

    @functools.partial(
        pl.kernel,
        mesh=mesh,
        out_type=jax.ShapeDtypeStruct((N, EMBED_DIM), jnp.float32),
        scratch_types=[
            pltpu.VMEM((J_TOTAL, IDX_LANE), jnp.int32),
            pltpu.VMEM((CHUNK_ROWS, EMBED_DIM), jnp.float32),
            pltpu.SemaphoreType.DMA,
        ],
        compiler_params=pltpu.CompilerParams(use_tc_tiling_on_sc=False),
    )
    def gather(idx_hbm, table_hbm, out_hbm, idx_v, rows_v, sem):
        wid = lax.axis_index("s") * NUM_CORES + lax.axis_index("c")
        base = wid * N_PER_W
        pltpu.sync_copy(idx_hbm.at[pl.ds(wid * J_TOTAL, J_TOTAL)], idx_v)

        def chunk_body(c, _):
            copies = []
            for j in range(J_PER_CHUNK):
                cp = pltpu.async_copy(
                    table_hbm.at[idx_v.at[c * J_PER_CHUNK + j]],
                    rows_v.at[pl.ds(j * IDX_LANE, IDX_LANE)],
                    sem,
                )
                copies.append(cp)
            for cp in copies:
                cp.wait()
            pltpu.sync_copy(
                rows_v,
                out_hbm.at[pl.ds(base + c * CHUNK_ROWS, CHUNK_ROWS)],
            )
            return 0

        lax.fori_loop(0, N_CHUNKS, chunk_body, 0)

    return gather


_gather = _make_gather()


ROW_BLK = 4096


def _proj_body(emb_ref, wt_ref, b_ref, out_ref):
    out_ref[...] = (
        jnp.dot(emb_ref[...], wt_ref[...], preferred_element_type=jnp.float32)
        + b_ref[...]
    )


def _project(emb, wt, b2d):
    return pl.pallas_call(
        _proj_body,
        grid=(N // ROW_BLK,),
        in_specs=[
            pl.BlockSpec((ROW_BLK, EMBED_DIM), lambda i: (i, 0)),
            pl.BlockSpec((EMBED_DIM, OUTPUT_DIM), lambda i: (0, 0)),
            pl.BlockSpec((1, OUTPUT_DIM), lambda i: (0, 0)),
        ],
        out_specs=pl.BlockSpec((ROW_BLK, OUTPUT_DIM), lambda i: (i, 0)),
        out_shape=jax.ShapeDtypeStruct((N, OUTPUT_DIM), jnp.float32),
    )(emb, wt, b2d)


def kernel(x, table, W, b):
    idx = x.reshape(N // IDX_LANE, IDX_LANE).astype(jnp.int32)
    emb = _gather(idx, table)
    out = _project(emb, W.T, b.reshape(1, OUTPUT_DIM))
    return out.reshape(BATCH, FIELDS, OUTPUT_DIM)


# speedup vs baseline: 18.0562x; 1.9148x over previous
import functools

import jax
import jax.numpy as jnp
from jax import lax
from jax.experimental import pallas as pl
from jax.experimental.pallas import tpu as pltpu
from jax.experimental.pallas import tpu_sc as plsc

BATCH = 16384
FIELDS = 26
EMBED_DIM = 32
OUTPUT_DIM = 64
VOCAB = 1000000
N = BATCH * FIELDS
N4 = N // 4
QUART = BATCH // 4

NUM_CORES = 2
NUM_SUBCORES = 16
NUM_WORKERS = NUM_CORES * NUM_SUBCORES
N_PER_W = N // NUM_WORKERS

IDX_LANE = 128
J_TOTAL = N_PER_W // IDX_LANE
J_PER_CHUNK = 8
CHUNK_ROWS = J_PER_CHUNK * IDX_LANE
N_CHUNKS = J_TOTAL // J_PER_CHUNK
CHUNKS_PER_FIELD = BATCH // CHUNK_ROWS
CHUNKS_PER_QUART = QUART // CHUNK_ROWS


def _make_gather():
    mesh = plsc.VectorSubcoreMesh(core_axis_name="c", subcore_axis_name="s")

    @functools.partial(
        pl.kernel,
        mesh=mesh,
        out_type=jax.ShapeDtypeStruct((FIELDS * QUART, 4 * EMBED_DIM), jnp.float32),
        scratch_types=[
            pltpu.VMEM((J_TOTAL, IDX_LANE), jnp.int32),
            pltpu.VMEM((CHUNK_ROWS, EMBED_DIM), jnp.float32),
            pltpu.SemaphoreType.DMA,
        ],
        compiler_params=pltpu.CompilerParams(use_tc_tiling_on_sc=False),
    )
    def gather(idx_hbm, table_hbm, out_hbm, idx_v, rows_v, sem):
        wid = lax.axis_index("s") * NUM_CORES + lax.axis_index("c")
        pltpu.sync_copy(idx_hbm.at[pl.ds(wid * J_TOTAL, J_TOTAL)], idx_v)

        def chunk_body(c, _):
            copies = []
            for j in range(J_PER_CHUNK):
                cp = pltpu.async_copy(
                    table_hbm.at[idx_v.at[c * J_PER_CHUNK + j]],
                    rows_v.at[pl.ds(j * IDX_LANE, IDX_LANE)],
                    sem,
                )
                copies.append(cp)
            for cp in copies:
                cp.wait()
            ck = wid * N_CHUNKS + c
            f = ck // CHUNKS_PER_FIELD
            r = ck % CHUNKS_PER_FIELD
            q = r // CHUNKS_PER_QUART
            row0 = f * QUART + (r % CHUNKS_PER_QUART) * CHUNK_ROWS
            pltpu.sync_copy(
                rows_v,
                out_hbm.at[pl.ds(row0, CHUNK_ROWS),
                           pl.ds(q * EMBED_DIM, EMBED_DIM)],
            )
            return 0

        lax.fori_loop(0, N_CHUNKS, chunk_body, 0)

    return gather


_gather = _make_gather()


def _proj_body(emb_ref, w_ref, b_ref, out_ref):
    w = w_ref[...]
    bb = b_ref[...]
    emb = emb_ref[0]
    for q in range(4):
        p = emb[:, q * EMBED_DIM:(q + 1) * EMBED_DIM]
        y = jax.lax.dot_general(
            w, p, (((1,), (1,)), ((), ())),
            preferred_element_type=jnp.float32,
        )
        out_ref[0, :, q * QUART:(q + 1) * QUART] = y + bb


def _project(emb3, w, b2):
    return pl.pallas_call(
        _proj_body,
        grid=(FIELDS,),
        in_specs=[
            pl.BlockSpec((1, QUART, 4 * EMBED_DIM), lambda f: (f, 0, 0)),
            pl.BlockSpec((OUTPUT_DIM, EMBED_DIM), lambda f: (0, 0)),
            pl.BlockSpec((OUTPUT_DIM, 1), lambda f: (0, 0)),
        ],
        out_specs=pl.BlockSpec((1, OUTPUT_DIM, BATCH), lambda f: (f, 0, 0)),
        out_shape=jax.ShapeDtypeStruct((FIELDS, OUTPUT_DIM, BATCH), jnp.float32),
    )(emb3, w, b2)


def kernel(x, table, W, b):
    idx = x.T.reshape(N // IDX_LANE, IDX_LANE).astype(jnp.int32)
    emb4 = _gather(idx, table)
    emb3 = emb4.reshape(FIELDS, QUART, 4 * EMBED_DIM)
    out_t = _project(emb3, W, b.reshape(OUTPUT_DIM, 1))
    return jnp.transpose(out_t, (2, 0, 1))


# speedup vs baseline: 29.5117x; 1.6344x over previous
import functools

import jax
import jax.numpy as jnp
from jax import lax
from jax.experimental import pallas as pl
from jax.experimental.pallas import tpu as pltpu
from jax.experimental.pallas import tpu_sc as plsc

BATCH = 16384
FIELDS = 26
EMBED_DIM = 32
OUTPUT_DIM = 64
VOCAB = 1000000
N = BATCH * FIELDS
N4 = N // 4
QUART = BATCH // 4

NUM_CORES = 2
NUM_SUBCORES = 16
NUM_WORKERS = NUM_CORES * NUM_SUBCORES
N_PER_W = N // NUM_WORKERS

IDX_LANE = 128
J_TOTAL = N_PER_W // IDX_LANE
J_PER_CHUNK = 8
CHUNK_ROWS = J_PER_CHUNK * IDX_LANE
N_CHUNKS = J_TOTAL // J_PER_CHUNK
CHUNKS_PER_FIELD = BATCH // CHUNK_ROWS
CHUNKS_PER_QUART = QUART // CHUNK_ROWS


TBLK = 2048
VBLK = 4 * TBLK
NBLK = (VOCAB + VBLK - 1) // VBLK
TPAD = NBLK * TBLK
VPAD = 4 * TPAD


def _fmt_body(t_ref, out_ref):
    eye = (
        lax.broadcasted_iota(jnp.int32, (EMBED_DIM, EMBED_DIM), 0)
        == lax.broadcasted_iota(jnp.int32, (EMBED_DIM, EMBED_DIM), 1)
    ).astype(jnp.float32)
    parts = []
    for k in range(4):
        tk = t_ref[:, k * TBLK:(k + 1) * TBLK]
        parts.append(jax.lax.dot_general(
            tk, eye, (((0,), (0,)), ((), ())),
            preferred_element_type=jnp.float32,
        ))
    out_ref[...] = jnp.concatenate(parts, axis=1)


def _format_table(table_t):
    return pl.pallas_call(
        _fmt_body,
        grid=(NBLK,),
        in_specs=[pl.BlockSpec((EMBED_DIM, VBLK), lambda i: (0, i))],
        out_specs=pl.BlockSpec((TBLK, 4 * EMBED_DIM), lambda i: (i, 0)),
        out_shape=jax.ShapeDtypeStruct((TPAD, 4 * EMBED_DIM), jnp.float32),
    )(table_t)


def _make_gather():
    mesh = plsc.VectorSubcoreMesh(core_axis_name="c", subcore_axis_name="s")

    @functools.partial(
        pl.kernel,
        mesh=mesh,
        out_type=jax.ShapeDtypeStruct((FIELDS * QUART, 4 * EMBED_DIM), jnp.float32),
        name="emb_gather",
        scratch_types=[
            pltpu.VMEM((J_TOTAL, IDX_LANE), jnp.int32),
            pltpu.VMEM((CHUNK_ROWS, EMBED_DIM), jnp.float32),
            pltpu.SemaphoreType.DMA,
        ],
        compiler_params=pltpu.CompilerParams(use_tc_tiling_on_sc=False),
    )
    def gather(idx_hbm, table_hbm, out_hbm, idx_v, rows_v, sem):
        wid = lax.axis_index("s") * NUM_CORES + lax.axis_index("c")
        pltpu.sync_copy(idx_hbm.at[pl.ds(wid * J_TOTAL, J_TOTAL)], idx_v)

        def remap_row(r, _):
            for l in range(IDX_LANE // 16):
                v = idx_v[r, pl.ds(l * 16, 16)]
                vv = v & (VBLK - 1)
                m = (v - vv) + ((vv & (TBLK - 1)) << 2) + (vv >> 11)
                idx_v[r, pl.ds(l * 16, 16)] = m
            return 0

        lax.fori_loop(0, J_TOTAL, remap_row, 0)

        def chunk_body(c, _):
            copies = []
            for j in range(J_PER_CHUNK):
                cp = pltpu.async_copy(
                    table_hbm.at[idx_v.at[c * J_PER_CHUNK + j]],
                    rows_v.at[pl.ds(j * IDX_LANE, IDX_LANE)],
                    sem,
                )
                copies.append(cp)
            for cp in copies:
                cp.wait()
            ck = wid * N_CHUNKS + c
            f = ck // CHUNKS_PER_FIELD
            r = ck % CHUNKS_PER_FIELD
            q = r // CHUNKS_PER_QUART
            row0 = f * QUART + (r % CHUNKS_PER_QUART) * CHUNK_ROWS
            pltpu.sync_copy(
                rows_v,
                out_hbm.at[pl.ds(row0, CHUNK_ROWS),
                           pl.ds(q * EMBED_DIM, EMBED_DIM)],
            )
            return 0

        lax.fori_loop(0, N_CHUNKS, chunk_body, 0)

    return gather


_gather = _make_gather()


def _proj_body(emb_ref, w_ref, b_ref, out_ref):
    w = w_ref[...]
    bb = b_ref[...]
    emb = emb_ref[0]
    for q in range(4):
        p = emb[:, q * EMBED_DIM:(q + 1) * EMBED_DIM]
        y = jax.lax.dot_general(
            w, p, (((1,), (1,)), ((), ())),
            preferred_element_type=jnp.float32,
        )
        out_ref[0, :, q * QUART:(q + 1) * QUART] = y + bb


def _project(emb3, w, b2):
    return pl.pallas_call(
        _proj_body,
        grid=(FIELDS,),
        in_specs=[
            pl.BlockSpec((1, QUART, 4 * EMBED_DIM), lambda f: (f, 0, 0)),
            pl.BlockSpec((OUTPUT_DIM, EMBED_DIM), lambda f: (0, 0)),
            pl.BlockSpec((OUTPUT_DIM, 1), lambda f: (0, 0)),
        ],
        out_specs=pl.BlockSpec((1, OUTPUT_DIM, BATCH), lambda f: (f, 0, 0)),
        out_shape=jax.ShapeDtypeStruct((FIELDS, OUTPUT_DIM, BATCH), jnp.float32),
    )(emb3, w, b2)


def kernel(x, table, W, b):
    idx = x.T.reshape(N // IDX_LANE, IDX_LANE).astype(jnp.int32)
    table4p = _format_table(table.T)
    emb4 = _gather(idx, table4p.reshape(VPAD, EMBED_DIM))
    emb3 = emb4.reshape(FIELDS, QUART, 4 * EMBED_DIM)
    out_t = _project(emb3, W, b.reshape(OUTPUT_DIM, 1))
    return jnp.transpose(out_t, (2, 0, 1))


# speedup vs baseline: 29.8070x; 1.0100x over previous
import functools

import jax
import jax.numpy as jnp
from jax import lax
from jax.experimental import pallas as pl
from jax.experimental.pallas import tpu as pltpu
from jax.experimental.pallas import tpu_sc as plsc

BATCH = 16384
FIELDS = 26
EMBED_DIM = 32
OUTPUT_DIM = 64
VOCAB = 1000000
N = BATCH * FIELDS
N4 = N // 4
QUART = BATCH // 4

NUM_CORES = 2
NUM_SUBCORES = 16
NUM_WORKERS = NUM_CORES * NUM_SUBCORES
N_PER_W = N // NUM_WORKERS

IDX_LANE = 128
J_TOTAL = N_PER_W // IDX_LANE
J_PER_CHUNK = 8
CHUNK_ROWS = J_PER_CHUNK * IDX_LANE
N_CHUNKS = J_TOTAL // J_PER_CHUNK
CHUNKS_PER_FIELD = BATCH // CHUNK_ROWS
CHUNKS_PER_QUART = QUART // CHUNK_ROWS


TBLK = 4096
VBLK = 4 * TBLK
NBLK = (VOCAB + VBLK - 1) // VBLK
TPAD = NBLK * TBLK
VPAD = 4 * TPAD


def _fmt_body(t_ref, out_ref):
    parts = []
    for k in range(4):
        tk = t_ref[:, k * TBLK:(k + 1) * TBLK]
        parts.append(jnp.transpose(tk))
    out_ref[...] = jnp.concatenate(parts, axis=1)


def _format_table(table_t):
    return pl.pallas_call(
        _fmt_body,
        grid=(NBLK,),
        in_specs=[pl.BlockSpec((EMBED_DIM, VBLK), lambda i: (0, i))],
        out_specs=pl.BlockSpec((TBLK, 4 * EMBED_DIM), lambda i: (i, 0)),
        out_shape=jax.ShapeDtypeStruct((TPAD, 4 * EMBED_DIM), jnp.float32),
        compiler_params=pltpu.CompilerParams(fuse_transposed_lhs_in_matmul=True),
    )(table_t)


def _make_gather():
    mesh = plsc.VectorSubcoreMesh(core_axis_name="c", subcore_axis_name="s")

    @functools.partial(
        pl.kernel,
        mesh=mesh,
        out_type=jax.ShapeDtypeStruct((FIELDS * QUART, 4 * EMBED_DIM), jnp.float32),
        name="emb_gather",
        scratch_types=[
            pltpu.VMEM((J_TOTAL, IDX_LANE), jnp.int32),
            pltpu.VMEM((CHUNK_ROWS, EMBED_DIM), jnp.float32),
            pltpu.SemaphoreType.DMA,
        ],
        compiler_params=pltpu.CompilerParams(use_tc_tiling_on_sc=False),
    )
    def gather(idx_hbm, table_hbm, out_hbm, idx_v, rows_v, sem):
        wid = lax.axis_index("s") * NUM_CORES + lax.axis_index("c")
        pltpu.sync_copy(idx_hbm.at[pl.ds(wid * J_TOTAL, J_TOTAL)], idx_v)

        def remap_row(r, _):
            for l in range(IDX_LANE // 16):
                v = idx_v[r, pl.ds(l * 16, 16)]
                vv = v & (VBLK - 1)
                m = (v - vv) + ((vv & (TBLK - 1)) << 2) + (vv >> (TBLK.bit_length() - 1))
                idx_v[r, pl.ds(l * 16, 16)] = m
            return 0

        lax.fori_loop(0, J_TOTAL, remap_row, 0)

        def chunk_body(c, _):
            copies = []
            for j in range(J_PER_CHUNK):
                cp = pltpu.async_copy(
                    table_hbm.at[idx_v.at[c * J_PER_CHUNK + j]],
                    rows_v.at[pl.ds(j * IDX_LANE, IDX_LANE)],
                    sem,
                )
                copies.append(cp)
            for cp in copies:
                cp.wait()
            ck = wid * N_CHUNKS + c
            f = ck // CHUNKS_PER_FIELD
            r = ck % CHUNKS_PER_FIELD
            q = r // CHUNKS_PER_QUART
            row0 = f * QUART + (r % CHUNKS_PER_QUART) * CHUNK_ROWS
            pltpu.sync_copy(
                rows_v,
                out_hbm.at[pl.ds(row0, CHUNK_ROWS),
                           pl.ds(q * EMBED_DIM, EMBED_DIM)],
            )
            return 0

        lax.fori_loop(0, N_CHUNKS, chunk_body, 0)

    return gather


_gather = _make_gather()


def _proj_body(emb_ref, w_ref, b_ref, out_ref):
    w = w_ref[...]
    bb = b_ref[...]
    emb = emb_ref[0]
    for q in range(4):
        p = emb[:, q * EMBED_DIM:(q + 1) * EMBED_DIM]
        y = jax.lax.dot_general(
            w, p, (((1,), (1,)), ((), ())),
            preferred_element_type=jnp.float32,
        )
        out_ref[0, :, q * QUART:(q + 1) * QUART] = y + bb


def _project(emb3, w, b2):
    return pl.pallas_call(
        _proj_body,
        grid=(FIELDS,),
        in_specs=[
            pl.BlockSpec((1, QUART, 4 * EMBED_DIM), lambda f: (f, 0, 0)),
            pl.BlockSpec((OUTPUT_DIM, EMBED_DIM), lambda f: (0, 0)),
            pl.BlockSpec((OUTPUT_DIM, 1), lambda f: (0, 0)),
        ],
        out_specs=pl.BlockSpec((1, OUTPUT_DIM, BATCH), lambda f: (f, 0, 0)),
        out_shape=jax.ShapeDtypeStruct((FIELDS, OUTPUT_DIM, BATCH), jnp.float32),
    )(emb3, w, b2)


def kernel(x, table, W, b):
    idx = x.T.reshape(N // IDX_LANE, IDX_LANE).astype(jnp.int32)
    table4p = _format_table(table.T)
    emb4 = _gather(idx, table4p.reshape(VPAD, EMBED_DIM))
    emb3 = emb4.reshape(FIELDS, QUART, 4 * EMBED_DIM)
    out_t = _project(emb3, W, b.reshape(OUTPUT_DIM, 1))
    return jnp.transpose(out_t, (2, 0, 1))
